# baseline (speedup 1.0000x reference)
import jax
import jax.numpy as jnp
from jax.experimental import pallas as pl


def _split_body(x_ref, m_ref, t_ref, v_ref, s_ref):
    xv = x_ref[...]
    m_ref[...] = xv[:, 0:32]
    t_ref[...] = xv[:, 32:64]
    v_ref[...] = xv[:, 64:96]
    s_ref[...] = xv[:, 96:128]


def kernel(x):
    B, S, F = x.shape
    R = B * S
    x2 = x.reshape(R, F)
    blk = 2048
    grid = (R // blk,)
    outs = pl.pallas_call(
        _split_body,
        grid=grid,
        in_specs=[pl.BlockSpec((blk, F), lambda i: (i, 0))],
        out_specs=tuple(pl.BlockSpec((blk, 32), lambda i: (i, 0)) for _ in range(4)),
        out_shape=tuple(jax.ShapeDtypeStruct((R, 32), x.dtype) for _ in range(4)),
    )(x2)
    return tuple(o.reshape(B, S, 32) for o in outs)
